# Initial kernel scaffold; baseline (speedup 1.0000x reference)
import jax
import jax.numpy as jnp
from jax.experimental import pallas as pl


def kernel(token_ids, E):
    raise NotImplementedError("write your pallas kernel here")


# speedup vs baseline: 1.0478x; 1.0478x over previous
import functools

import jax
import jax.numpy as jnp
from jax import lax
from jax.experimental import pallas as pl
from jax.experimental.pallas import tpu as pltpu
from jax.experimental.pallas import tpu_sc as plsc

_DIM = 64
_NUM_WORKERS = 32
_WINDOW = 512
_TC_BLOCK = 2048


def kernel(token_ids, E):
    B, L = token_ids.shape
    n = B * L
    tok = token_ids.reshape(n).astype(jnp.int32)
    pair_idx = tok >> 1
    parity = (tok & 1).reshape(n, 1)
    num_emb = E.shape[0]
    E2 = E.reshape(num_emb // 2, 2 * _DIM)

    b_per_w = n // _NUM_WORKERS
    n_windows = b_per_w // _WINDOW
    mesh = plsc.VectorSubcoreMesh(core_axis_name="c", subcore_axis_name="s")

    @functools.partial(
        pl.kernel,
        mesh=mesh,
        out_type=jax.ShapeDtypeStruct((n, 2 * _DIM), jnp.float32),
        scratch_types=[
            pltpu.VMEM((_WINDOW,), jnp.int32),
            pltpu.VMEM((_WINDOW, 2 * _DIM), jnp.float32),
            pltpu.SemaphoreType.DMA,
        ],
    )
    def gather_kernel(table_hbm, idx_hbm, out_hbm, idx_v, rows_v, sem):
        wid = lax.axis_index("s") * 2 + lax.axis_index("c")
        base = wid * b_per_w

        @pl.loop(0, n_windows)
        def _(g):
            off = base + g * _WINDOW
            pltpu.sync_copy(idx_hbm.at[pl.ds(off, _WINDOW)], idx_v)
            pltpu.async_copy(table_hbm.at[idx_v], rows_v, sem).wait()
            pltpu.sync_copy(rows_v, out_hbm.at[pl.ds(off, _WINDOW)])

    pairs = gather_kernel(E2, pair_idx)

    def select_body(g_ref, p_ref, o_ref):
        p = p_ref[...]
        o_ref[...] = jnp.where(p == 1, g_ref[:, _DIM:], g_ref[:, :_DIM])

    out = pl.pallas_call(
        select_body,
        grid=(n // _TC_BLOCK,),
        in_specs=[
            pl.BlockSpec((_TC_BLOCK, 2 * _DIM), lambda i: (i, 0)),
            pl.BlockSpec((_TC_BLOCK, 1), lambda i: (i, 0)),
        ],
        out_specs=pl.BlockSpec((_TC_BLOCK, _DIM), lambda i: (i, 0)),
        out_shape=jax.ShapeDtypeStruct((n, _DIM), jnp.float32),
        compiler_params=pltpu.CompilerParams(
            dimension_semantics=("parallel",),
        ),
    )(pairs, parity)

    return out.reshape(B, L, _DIM)


# speedup vs baseline: 1.4498x; 1.3837x over previous
import functools

import jax
import jax.numpy as jnp
from jax import lax
from jax.experimental import pallas as pl
from jax.experimental.pallas import tpu as pltpu
from jax.experimental.pallas import tpu_sc as plsc

_DIM = 64
_NUM_WORKERS = 32
_W = 256


def kernel(token_ids, E):
    B, L = token_ids.shape
    n = B * L
    tok = token_ids.reshape(n).astype(jnp.int32)
    num_emb = E.shape[0]
    Edup = jnp.concatenate([E, E], axis=1)

    b_per_w = n // _NUM_WORKERS
    n_windows = b_per_w // _W
    mesh = plsc.VectorSubcoreMesh(core_axis_name="c", subcore_axis_name="s")

    @functools.partial(
        pl.kernel,
        mesh=mesh,
        out_type=jax.ShapeDtypeStruct((n // 2, 2 * _DIM), jnp.float32),
        scratch_types=[
            pltpu.VMEM((_W,), jnp.int32),
            pltpu.VMEM((_W,), jnp.int32),
            pltpu.VMEM((_W, 2 * _DIM), jnp.float32),
            pltpu.VMEM((_W, 2 * _DIM), jnp.float32),
            pltpu.VMEM((_W // 2, 2 * _DIM), jnp.float32),
            pltpu.VMEM((_W // 2, 2 * _DIM), jnp.float32),
            pltpu.SemaphoreType.DMA,
            pltpu.SemaphoreType.DMA,
            pltpu.SemaphoreType.DMA,
            pltpu.SemaphoreType.DMA,
        ],
    )
    def gather_kernel(table_hbm, tok_hbm, out_hbm,
                      tok0, tok1, rows0, rows1, outd0, outd1,
                      sg0, sg1, so0, so1):
        tokv = (tok0, tok1)
        rows = (rows0, rows1)
        outd = (outd0, outd1)
        sg = (sg0, sg1)
        so = (so0, so1)

        wid = lax.axis_index("s") * 2 + lax.axis_index("c")
        base = wid * b_per_w

        def start(g, b):
            off = pl.multiple_of(base + g * _W, _W)
            pltpu.sync_copy(tok_hbm.at[pl.ds(off, _W)], tokv[b])
            pltpu.async_copy(table_hbm.at[tokv[b]], rows[b], sg[b])

        def finish(g, b):
            off2 = pl.multiple_of((base + g * _W) // 2, _W // 2)
            pltpu.make_async_copy(table_hbm.at[tokv[b]], rows[b], sg[b]).wait()

            @pl.when(g >= 2)
            def _():
                pltpu.make_async_copy(
                    outd[b], out_hbm.at[pl.ds(0, _W // 2)], so[b]
                ).wait()

            @pl.loop(0, _W // 2)
            def _(j):
                w = j * 2
                for c in range(0, _DIM, 16):
                    outd[b][j, pl.ds(c, 16)] = rows[b][w, pl.ds(c, 16)]
                for c in range(0, _DIM, 16):
                    outd[b][j, pl.ds(_DIM + c, 16)] = rows[b][w + 1, pl.ds(c, 16)]

            pltpu.async_copy(outd[b], out_hbm.at[pl.ds(off2, _W // 2)], so[b])

        start(0, 0)
        start(1, 1)

        @pl.loop(0, n_windows, step=2)
        def _(g):
            finish(g, 0)

            @pl.when(g + 2 < n_windows)
            def _():
                start(g + 2, 0)

            finish(g + 1, 1)

            @pl.when(g + 3 < n_windows)
            def _():
                start(g + 3, 1)

        pltpu.make_async_copy(
            outd[0], out_hbm.at[pl.ds(0, _W // 2)], so[0]
        ).wait()
        pltpu.make_async_copy(
            outd[1], out_hbm.at[pl.ds(0, _W // 2)], so[1]
        ).wait()

    out2 = gather_kernel(Edup, tok)
    return out2.reshape(B, L, _DIM)


# speedup vs baseline: 1.8442x; 1.2721x over previous
import functools

import jax
import jax.numpy as jnp
from jax import lax
from jax.experimental import pallas as pl
from jax.experimental.pallas import tpu as pltpu
from jax.experimental.pallas import tpu_sc as plsc

_DIM = 64
_NUM_WORKERS = 32
_W = 256
_NBUF = 4


def kernel(token_ids, E):
    B, L = token_ids.shape
    n = B * L
    tok = token_ids.reshape(n).astype(jnp.int32)

    b_per_w = n // _NUM_WORKERS
    n_windows = b_per_w // _W
    mesh = plsc.VectorSubcoreMesh(core_axis_name="c", subcore_axis_name="s")

    @functools.partial(
        pl.kernel,
        mesh=mesh,
        out_type=jax.ShapeDtypeStruct((n, _DIM), jnp.float32),
        scratch_types=(
            [pltpu.VMEM((_W,), jnp.int32) for _ in range(_NBUF)]
            + [pltpu.VMEM((_W, _DIM), jnp.float32) for _ in range(_NBUF)]
            + [pltpu.SemaphoreType.DMA for _ in range(2 * _NBUF)]
        ),
        compiler_params=pltpu.CompilerParams(use_tc_tiling_on_sc=False),
    )
    def gather_kernel(table_hbm, tok_hbm, out_hbm, *scratch):
        tokv = scratch[:_NBUF]
        rows = scratch[_NBUF:2 * _NBUF]
        sg = scratch[2 * _NBUF:3 * _NBUF]
        so = scratch[3 * _NBUF:4 * _NBUF]

        wid = lax.axis_index("s") * 2 + lax.axis_index("c")
        base = wid * b_per_w

        def start(g, b):
            off = pl.multiple_of(base + g * _W, _W)

            @pl.when(g >= _NBUF)
            def _():
                pltpu.make_async_copy(
                    rows[b], out_hbm.at[pl.ds(0, _W)], so[b]
                ).wait()

            pltpu.sync_copy(tok_hbm.at[pl.ds(off, _W)], tokv[b])
            pltpu.async_copy(table_hbm.at[tokv[b]], rows[b], sg[b])

        def finish(g, b):
            off = pl.multiple_of(base + g * _W, _W)
            pltpu.make_async_copy(table_hbm.at[tokv[b]], rows[b], sg[b]).wait()
            pltpu.async_copy(rows[b], out_hbm.at[pl.ds(off, _W)], so[b])

        for b in range(_NBUF):
            start(b, b)

        @pl.loop(0, n_windows, step=_NBUF)
        def _(g):
            for d in range(_NBUF):
                finish(g + d, d)

                @pl.when(g + d + _NBUF < n_windows)
                def _():
                    start(g + d + _NBUF, d)

        for b in range(_NBUF):
            pltpu.make_async_copy(
                rows[b], out_hbm.at[pl.ds(0, _W)], so[b]
            ).wait()

    out = gather_kernel(E, tok)
    return out.reshape(B, L, _DIM)


# speedup vs baseline: 1.8691x; 1.0135x over previous
import functools

import jax
import jax.numpy as jnp
from jax import lax
from jax.experimental import pallas as pl
from jax.experimental.pallas import tpu as pltpu
from jax.experimental.pallas import tpu_sc as plsc

_DIM = 64
_NUM_WORKERS = 32
_W = 256
_NBUF = 3


def kernel(token_ids, E):
    B, L = token_ids.shape
    n = B * L
    tok = token_ids.reshape(n).astype(jnp.int32)

    b_per_w = n // _NUM_WORKERS
    n_windows = b_per_w // _W
    mesh = plsc.VectorSubcoreMesh(core_axis_name="c", subcore_axis_name="s")

    @functools.partial(
        pl.kernel,
        mesh=mesh,
        out_type=jax.ShapeDtypeStruct((n // 2, 2 * _DIM), jnp.float32),
        scratch_types=(
            [pltpu.VMEM((_W,), jnp.int32) for _ in range(_NBUF)]
            + [pltpu.VMEM((_W, _DIM), jnp.float32) for _ in range(_NBUF)]
            + [pltpu.VMEM((_W // 2, 2 * _DIM), jnp.float32) for _ in range(_NBUF)]
            + [pltpu.SemaphoreType.DMA for _ in range(2 * _NBUF)]
        ),
        compiler_params=pltpu.CompilerParams(use_tc_tiling_on_sc=False),
    )
    def gather_kernel(table_hbm, tok_hbm, out_hbm, *scratch):
        tokv = scratch[:_NBUF]
        rows = scratch[_NBUF:2 * _NBUF]
        outd = scratch[2 * _NBUF:3 * _NBUF]
        sg = scratch[3 * _NBUF:4 * _NBUF]
        so = scratch[4 * _NBUF:5 * _NBUF]

        wid = lax.axis_index("s") * 2 + lax.axis_index("c")
        base = wid * b_per_w

        def start(g, b):
            off = pl.multiple_of(base + g * _W, _W)
            pltpu.sync_copy(tok_hbm.at[pl.ds(off, _W)], tokv[b])
            pltpu.async_copy(table_hbm.at[tokv[b]], rows[b], sg[b])

        def finish(g, b):
            off2 = pl.multiple_of((base + g * _W) // 2, _W // 2)
            pltpu.make_async_copy(table_hbm.at[tokv[b]], rows[b], sg[b]).wait()

            @pl.when(g >= _NBUF)
            def _():
                pltpu.make_async_copy(
                    outd[b], out_hbm.at[pl.ds(0, _W // 2)], so[b]
                ).wait()

            @pl.loop(0, _W // 2)
            def _(j):
                w = j * 2
                for c in range(0, _DIM, 16):
                    outd[b][j, pl.ds(c, 16)] = rows[b][w, pl.ds(c, 16)]
                for c in range(0, _DIM, 16):
                    outd[b][j, pl.ds(_DIM + c, 16)] = rows[b][w + 1, pl.ds(c, 16)]

            pltpu.async_copy(outd[b], out_hbm.at[pl.ds(off2, _W // 2)], so[b])

        for b in range(_NBUF):
            start(b, b)

        @pl.loop(0, n_windows, step=_NBUF)
        def _(g):
            for d in range(_NBUF):
                @pl.when(g + d < n_windows)
                def _():
                    finish(g + d, d)

                @pl.when(g + d + _NBUF < n_windows)
                def _():
                    start(g + d + _NBUF, d)

        for b in range(_NBUF):
            pltpu.make_async_copy(
                outd[b], out_hbm.at[pl.ds(0, _W // 2)], so[b]
            ).wait()

    out2 = gather_kernel(E, tok)
    return out2.reshape(B, L, _DIM)
